# baseline (speedup 1.0000x reference)
import functools

import jax
import jax.numpy as jnp
from jax import lax
from jax.experimental import pallas as pl
from jax.experimental.pallas import tpu as pltpu
from jax.experimental.pallas import tpu_sc as plsc

D = 768
H = 3072
E = 8
K = 2

BLK_T = 256
HC = 2
HB = H // HC
CB = 512
INV_SQRT2 = 0.7071067811865476
HIGH = lax.Precision.HIGHEST


def _gelu(h):
    return 0.5 * h * (1.0 + lax.erf(h * INV_SQRT2))


def _router_body(x_ref, wg_ref, sp_ref, wc_ref, off_ref, aux_ref):
    x = x_ref[...]
    T = x.shape[0]
    T2 = K * T
    logits = jnp.dot(x, wg_ref[...], preferred_element_type=jnp.float32)
    m = jnp.max(logits, axis=-1, keepdims=True)
    ex = jnp.exp(logits - m)
    probs = ex / jnp.sum(ex, axis=-1, keepdims=True)
    iota = lax.broadcasted_iota(jnp.int32, probs.shape, 1)
    m0 = jnp.max(probs, axis=-1, keepdims=True)
    i0 = jnp.min(jnp.where(probs == m0, iota, E), axis=-1, keepdims=True)
    probs2 = jnp.where(iota == i0, -1.0, probs)
    m1 = jnp.max(probs2, axis=-1, keepdims=True)
    i1 = jnp.min(jnp.where(probs2 == m1, iota, E), axis=-1, keepdims=True)
    s = m0 + m1
    w0 = m0 / s
    w1 = m1 / s
    oh0 = (iota == i0).astype(jnp.float32)
    oh1 = (iota == i1).astype(jnp.float32)

    wc_ref[...] = jnp.where(iota == 0, w0, 0.0) + jnp.where(iota == 1, w1, 0.0)

    counts = jnp.sum(oh0 + oh1, axis=0, keepdims=True)
    p_i = jnp.sum(probs, axis=0, keepdims=True) / T
    aux_ref[0, 0] = E * jnp.sum((counts / T) * p_i)

    lane_r = lax.broadcasted_iota(jnp.int32, (E, E), 0)
    lane_c = lax.broadcasted_iota(jnp.int32, (E, E), 1)
    mtri = (lane_r < lane_c).astype(jnp.float32)
    offv = jnp.dot(counts, mtri, preferred_element_type=jnp.float32,
                   precision=HIGH)
    lane8 = lax.broadcasted_iota(jnp.int32, (1, E), 1)
    for e in range(E):
        off_ref[0, e] = jnp.sum(
            jnp.where(lane8 == e, offv, 0.0)).astype(jnp.int32)
    off_ref[0, E] = T2

    ohA = jnp.concatenate([oh0, oh1], axis=0)
    row_b = lax.broadcasted_iota(jnp.int32, (CB, CB), 0)
    col_b = lax.broadcasted_iota(jnp.int32, (CB, CB), 1)
    ltri = (row_b > col_b).astype(jnp.float32)
    carry = jnp.zeros((1, E), jnp.float32)
    pieces = []
    for bidx in range(T2 // CB):
        sub = lax.slice(ohA, (bidx * CB, 0), (bidx * CB + CB, E))
        loc = jnp.dot(ltri, sub, preferred_element_type=jnp.float32,
                      precision=HIGH) + carry
        pieces.append(loc)
        carry = carry + jnp.sum(sub, axis=0, keepdims=True)
    cum = jnp.concatenate(pieces, axis=0)
    spv = jnp.sum(ohA * (cum + offv), axis=-1, keepdims=True)
    sp_ref[...] = spv.astype(jnp.int32)


def _gffn_body(off_ref, xs_ref, w1_ref, b1_ref, w2_ref, b2_ref, y_ref,
               w1s_ref, w2s_ref):
    e = pl.program_id(0)
    hc = pl.program_id(1)
    w1s_ref[...] = w1_ref[0].astype(jnp.bfloat16)
    w2s_ref[...] = w2_ref[0].astype(jnp.bfloat16)
    lo_e = off_ref[0, e]
    hi_e = off_ref[0, e + 1]
    j0 = lo_e // BLK_T
    j1 = (hi_e + BLK_T - 1) // BLK_T

    def _block(j, carry):
        r0 = pl.multiple_of(j * BLK_T, BLK_T)
        lo = jnp.maximum(lo_e, r0)
        hi = jnp.minimum(hi_e, r0 + BLK_T)
        xb = xs_ref[pl.ds(r0, BLK_T), :].astype(jnp.bfloat16)
        h = jnp.dot(xb, w1s_ref[...], preferred_element_type=jnp.float32)
        h = _gelu(h + b1_ref[0, 0])
        y = jnp.dot(h.astype(jnp.bfloat16), w2s_ref[...],
                    preferred_element_type=jnp.float32)
        row = r0 + lax.broadcasted_iota(jnp.int32, (BLK_T, D), 0)
        mask = jnp.logical_and(row >= lo, row < hi)

        @pl.when(hc == 0)
        def _init():
            y_ref[pl.ds(r0, BLK_T), :] = jnp.where(
                mask, y + b2_ref[0, 0], y_ref[pl.ds(r0, BLK_T), :])

        @pl.when(hc != 0)
        def _acc():
            old = y_ref[pl.ds(r0, BLK_T), :]
            y_ref[pl.ds(r0, BLK_T), :] = jnp.where(mask, old + y, old)

        return carry

    lax.fori_loop(j0, j1, _block, 0)


def _combine_body(yg0_ref, yg1_ref, wc_ref, out_ref):
    wc = wc_ref[...]
    lane = lax.broadcasted_iota(jnp.int32, wc.shape, 1)
    w0 = jnp.sum(jnp.where(lane == 0, wc, 0.0), axis=-1, keepdims=True)
    w1 = jnp.sum(jnp.where(lane == 1, wc, 0.0), axis=-1, keepdims=True)
    out_ref[...] = (w0 * yg0_ref[...].astype(jnp.float32)
                    + w1 * yg1_ref[...].astype(jnp.float32))


def _sc_info():
    info = plsc.get_sparse_core_info()
    return info.num_cores, info.num_subcores


@functools.lru_cache(maxsize=None)
def _make_sc_kernels(T, T2, Dn):
    NC, NS = _sc_info()
    NW = NC * NS
    rows_per = T2 // NW
    mesh = plsc.VectorSubcoreMesh(core_axis_name="c", subcore_axis_name="s")
    scratch = [
        pltpu.VMEM((rows_per,), jnp.int32),
        pltpu.VMEM((rows_per, Dn), jnp.float32),
        pltpu.SemaphoreType.DMA,
    ]

    @functools.partial(
        pl.kernel, mesh=mesh,
        out_type=jax.ShapeDtypeStruct((T2, Dn), jnp.float32),
        scratch_types=scratch,
    )
    def dispatch(x_hbm, sp_hbm, xs_hbm, idx_v, rows_v, sem):
        wid = lax.axis_index("s") * NC + lax.axis_index("c")
        pltpu.sync_copy(sp_hbm.at[wid], idx_v)
        tok0 = lax.rem(wid * rows_per, T)
        pltpu.sync_copy(x_hbm.at[pl.ds(tok0, rows_per)], rows_v)
        pltpu.async_copy(rows_v, xs_hbm.at[idx_v], sem).wait()

    @functools.partial(
        pl.kernel, mesh=mesh,
        out_type=jax.ShapeDtypeStruct((T2, Dn), jnp.float32),
        scratch_types=scratch,
    )
    def gather(y_hbm, sp_hbm, yg_hbm, idx_v, rows_v, sem):
        wid = lax.axis_index("s") * NC + lax.axis_index("c")
        pltpu.sync_copy(sp_hbm.at[wid], idx_v)
        pltpu.async_copy(y_hbm.at[idx_v], rows_v, sem).wait()
        pltpu.sync_copy(rows_v, yg_hbm.at[pl.ds(wid * rows_per, rows_per)])

    return dispatch, gather, NW


def kernel(x, Wg, W1, b1, W2, b2):
    b, s, d = x.shape
    T = b * s
    T2 = K * T
    xf = x.reshape(T, d)

    sp, wc, off, aux = pl.pallas_call(
        _router_body,
        out_shape=(
            jax.ShapeDtypeStruct((T2, 1), jnp.int32),
            jax.ShapeDtypeStruct((T, E), jnp.float32),
            jax.ShapeDtypeStruct((1, 16), jnp.int32),
            jax.ShapeDtypeStruct((1, 1), jnp.float32),
        ),
        in_specs=[
            pl.BlockSpec(memory_space=pltpu.VMEM),
            pl.BlockSpec(memory_space=pltpu.VMEM),
        ],
        out_specs=(
            pl.BlockSpec(memory_space=pltpu.VMEM),
            pl.BlockSpec(memory_space=pltpu.VMEM),
            pl.BlockSpec(memory_space=pltpu.SMEM),
            pl.BlockSpec(memory_space=pltpu.SMEM),
        ),
    )(xf, Wg)

    dispatch, gather, NW = _make_sc_kernels(T, T2, d)
    sp_rows = sp.reshape(NW, T2 // NW)

    xs = dispatch(xf, sp_rows)

    b1r = b1.reshape(E, HC, 1, HB)
    b2r = b2.reshape(E, 1, D)

    grid = (E, HC)
    y = pl.pallas_call(
        _gffn_body,
        grid=grid,
        in_specs=[
            pl.BlockSpec(memory_space=pltpu.SMEM),
            pl.BlockSpec((T2, D), lambda e, hc: (0, 0)),
            pl.BlockSpec((1, D, HB), lambda e, hc: (e, 0, hc)),
            pl.BlockSpec((1, 1, 1, HB), lambda e, hc: (e, hc, 0, 0)),
            pl.BlockSpec((1, HB, D), lambda e, hc: (e, hc, 0)),
            pl.BlockSpec((1, 1, D), lambda e, hc: (e, 0, 0)),
        ],
        out_specs=pl.BlockSpec((T2, D), lambda e, hc: (0, 0)),
        out_shape=jax.ShapeDtypeStruct((T2, D), jnp.float32),
        scratch_shapes=[
            pltpu.VMEM((D, HB), jnp.bfloat16),
            pltpu.VMEM((HB, D), jnp.bfloat16),
        ],
        compiler_params=pltpu.CompilerParams(
            vmem_limit_bytes=110 * 1024 * 1024),
    )(off, xs, W1, b1r, W2, b2r)

    yg = gather(y, sp_rows)

    CT = 512
    out = pl.pallas_call(
        _combine_body,
        grid=(T // CT,),
        in_specs=[
            pl.BlockSpec((CT, D), lambda c: (c, 0)),
            pl.BlockSpec((CT, D), lambda c, nb=T // CT: (c + nb, 0)),
            pl.BlockSpec((CT, E), lambda c: (c, 0)),
        ],
        out_specs=pl.BlockSpec((CT, D), lambda c: (c, 0)),
        out_shape=jax.ShapeDtypeStruct((T, D), jnp.float32),
    )(yg, yg, wc)

    return out.reshape(b, s, d), aux.reshape(())


# speedup vs baseline: 1.0209x; 1.0209x over previous
import functools

import jax
import jax.numpy as jnp
from jax import lax
from jax.experimental import pallas as pl
from jax.experimental.pallas import tpu as pltpu
from jax.experimental.pallas import tpu_sc as plsc

D = 768
H = 3072
E = 8
K = 2

BLK_T = 256
HC = 2
HB = H // HC
CB = 256
INV_SQRT2 = 0.7071067811865476
HIGH = lax.Precision.HIGHEST


def _gelu(h):
    return 0.5 * h * (1.0 + lax.erf(h * INV_SQRT2))


def _router_body(x_ref, wg_ref, sp_ref, wc_ref, off_ref, aux_ref):
    x = x_ref[...]
    T = x.shape[0]
    T2 = K * T
    logits = jnp.dot(x, wg_ref[...], preferred_element_type=jnp.float32)
    m = jnp.max(logits, axis=-1, keepdims=True)
    ex = jnp.exp(logits - m)
    probs = ex / jnp.sum(ex, axis=-1, keepdims=True)
    iota = lax.broadcasted_iota(jnp.int32, probs.shape, 1)
    m0 = jnp.max(probs, axis=-1, keepdims=True)
    i0 = jnp.min(jnp.where(probs == m0, iota, E), axis=-1, keepdims=True)
    probs2 = jnp.where(iota == i0, -1.0, probs)
    m1 = jnp.max(probs2, axis=-1, keepdims=True)
    i1 = jnp.min(jnp.where(probs2 == m1, iota, E), axis=-1, keepdims=True)
    s = m0 + m1
    w0 = m0 / s
    w1 = m1 / s
    oh0 = (iota == i0).astype(jnp.float32)
    oh1 = (iota == i1).astype(jnp.float32)

    wc_ref[...] = jnp.where(iota == 0, w0, 0.0) + jnp.where(iota == 1, w1, 0.0)

    counts = jnp.sum(oh0 + oh1, axis=0, keepdims=True)
    p_i = jnp.sum(probs, axis=0, keepdims=True) / T
    aux_ref[0, 0] = E * jnp.sum((counts / T) * p_i)

    lane_r = lax.broadcasted_iota(jnp.int32, (E, E), 0)
    lane_c = lax.broadcasted_iota(jnp.int32, (E, E), 1)
    mtri = (lane_r < lane_c).astype(jnp.float32)
    offv = jnp.dot(counts, mtri, preferred_element_type=jnp.float32,
                   precision=HIGH)
    lane8 = lax.broadcasted_iota(jnp.int32, (1, E), 1)
    for e in range(E):
        off_ref[0, e] = jnp.sum(
            jnp.where(lane8 == e, offv, 0.0)).astype(jnp.int32)
    off_ref[0, E] = T2

    ohA = jnp.concatenate([oh0, oh1], axis=0)
    row_b = lax.broadcasted_iota(jnp.int32, (CB, CB), 0)
    col_b = lax.broadcasted_iota(jnp.int32, (CB, CB), 1)
    ltri = (row_b > col_b).astype(jnp.float32)
    carry = jnp.zeros((1, E), jnp.float32)
    pieces = []
    for bidx in range(T2 // CB):
        sub = lax.slice(ohA, (bidx * CB, 0), (bidx * CB + CB, E))
        loc = jnp.dot(ltri, sub, preferred_element_type=jnp.float32,
                      precision=HIGH) + carry
        pieces.append(loc)
        carry = carry + jnp.sum(sub, axis=0, keepdims=True)
    cum = jnp.concatenate(pieces, axis=0)
    spv = jnp.sum(ohA * (cum + offv), axis=-1, keepdims=True)
    sp_ref[...] = spv.astype(jnp.int32)


def _gffn_body(off_ref, xs_ref, w1_ref, b1_ref, w2_ref, b2_ref, y_ref,
               w1s_ref, w2s_ref):
    e = pl.program_id(0)
    hc = pl.program_id(1)
    w1s_ref[...] = w1_ref[0].astype(jnp.bfloat16)
    w2s_ref[...] = w2_ref[0].astype(jnp.bfloat16)
    lo_e = off_ref[0, e]
    hi_e = off_ref[0, e + 1]
    j0 = lo_e // BLK_T
    j1 = (hi_e + BLK_T - 1) // BLK_T

    def _block(j, carry):
        r0 = pl.multiple_of(j * BLK_T, BLK_T)
        lo = jnp.maximum(lo_e, r0)
        hi = jnp.minimum(hi_e, r0 + BLK_T)
        xb = xs_ref[pl.ds(r0, BLK_T), :].astype(jnp.bfloat16)
        h = jnp.dot(xb, w1s_ref[...], preferred_element_type=jnp.float32)
        h = _gelu(h + b1_ref[0, 0])
        y = jnp.dot(h.astype(jnp.bfloat16), w2s_ref[...],
                    preferred_element_type=jnp.float32)
        row = r0 + lax.broadcasted_iota(jnp.int32, (BLK_T, D), 0)
        mask = jnp.logical_and(row >= lo, row < hi)

        @pl.when(hc == 0)
        def _init():
            y_ref[pl.ds(r0, BLK_T), :] = jnp.where(
                mask, y + b2_ref[0, 0], y_ref[pl.ds(r0, BLK_T), :])

        @pl.when(hc != 0)
        def _acc():
            old = y_ref[pl.ds(r0, BLK_T), :]
            y_ref[pl.ds(r0, BLK_T), :] = jnp.where(mask, old + y, old)

        return carry

    lax.fori_loop(j0, j1, _block, 0)


def _combine_body(yg0_ref, yg1_ref, wc_ref, out_ref):
    wc = wc_ref[...]
    lane = lax.broadcasted_iota(jnp.int32, wc.shape, 1)
    w0 = jnp.sum(jnp.where(lane == 0, wc, 0.0), axis=-1, keepdims=True)
    w1 = jnp.sum(jnp.where(lane == 1, wc, 0.0), axis=-1, keepdims=True)
    out_ref[...] = (w0 * yg0_ref[...].astype(jnp.float32)
                    + w1 * yg1_ref[...].astype(jnp.float32))


def _sc_info():
    info = plsc.get_sparse_core_info()
    return info.num_cores, info.num_subcores


@functools.lru_cache(maxsize=None)
def _make_sc_kernels(T, T2, Dn):
    NC, NS = _sc_info()
    NW = NC * NS
    rows_per = T2 // NW
    mesh = plsc.VectorSubcoreMesh(core_axis_name="c", subcore_axis_name="s")
    scratch = [
        pltpu.VMEM((rows_per,), jnp.int32),
        pltpu.VMEM((rows_per, Dn), jnp.float32),
        pltpu.SemaphoreType.DMA,
    ]

    @functools.partial(
        pl.kernel, mesh=mesh,
        out_type=jax.ShapeDtypeStruct((T2, Dn), jnp.float32),
        scratch_types=scratch,
    )
    def dispatch(x_hbm, sp_hbm, xs_hbm, idx_v, rows_v, sem):
        wid = lax.axis_index("s") * NC + lax.axis_index("c")
        pltpu.sync_copy(sp_hbm.at[wid], idx_v)
        tok0 = lax.rem(wid * rows_per, T)
        pltpu.sync_copy(x_hbm.at[pl.ds(tok0, rows_per)], rows_v)
        pltpu.async_copy(rows_v, xs_hbm.at[idx_v], sem).wait()

    @functools.partial(
        pl.kernel, mesh=mesh,
        out_type=jax.ShapeDtypeStruct((T2, Dn), jnp.float32),
        scratch_types=scratch,
    )
    def gather(y_hbm, sp_hbm, yg_hbm, idx_v, rows_v, sem):
        wid = lax.axis_index("s") * NC + lax.axis_index("c")
        pltpu.sync_copy(sp_hbm.at[wid], idx_v)
        pltpu.async_copy(y_hbm.at[idx_v], rows_v, sem).wait()
        pltpu.sync_copy(rows_v, yg_hbm.at[pl.ds(wid * rows_per, rows_per)])

    return dispatch, gather, NW


def kernel(x, Wg, W1, b1, W2, b2):
    b, s, d = x.shape
    T = b * s
    T2 = K * T
    xf = x.reshape(T, d)

    sp, wc, off, aux = pl.pallas_call(
        _router_body,
        out_shape=(
            jax.ShapeDtypeStruct((T2, 1), jnp.int32),
            jax.ShapeDtypeStruct((T, E), jnp.float32),
            jax.ShapeDtypeStruct((1, 16), jnp.int32),
            jax.ShapeDtypeStruct((1, 1), jnp.float32),
        ),
        in_specs=[
            pl.BlockSpec(memory_space=pltpu.VMEM),
            pl.BlockSpec(memory_space=pltpu.VMEM),
        ],
        out_specs=(
            pl.BlockSpec(memory_space=pltpu.VMEM),
            pl.BlockSpec(memory_space=pltpu.VMEM),
            pl.BlockSpec(memory_space=pltpu.SMEM),
            pl.BlockSpec(memory_space=pltpu.SMEM),
        ),
    )(xf, Wg)

    dispatch, gather, NW = _make_sc_kernels(T, T2, d)
    sp_rows = sp.reshape(NW, T2 // NW)

    xs = dispatch(xf, sp_rows)

    b1r = b1.reshape(E, HC, 1, HB)
    b2r = b2.reshape(E, 1, D)

    grid = (E, HC)
    y = pl.pallas_call(
        _gffn_body,
        grid=grid,
        in_specs=[
            pl.BlockSpec(memory_space=pltpu.SMEM),
            pl.BlockSpec((T2, D), lambda e, hc: (0, 0)),
            pl.BlockSpec((1, D, HB), lambda e, hc: (e, 0, hc)),
            pl.BlockSpec((1, 1, 1, HB), lambda e, hc: (e, hc, 0, 0)),
            pl.BlockSpec((1, HB, D), lambda e, hc: (e, hc, 0)),
            pl.BlockSpec((1, 1, D), lambda e, hc: (e, 0, 0)),
        ],
        out_specs=pl.BlockSpec((T2, D), lambda e, hc: (0, 0)),
        out_shape=jax.ShapeDtypeStruct((T2, D), jnp.float32),
        scratch_shapes=[
            pltpu.VMEM((D, HB), jnp.bfloat16),
            pltpu.VMEM((HB, D), jnp.bfloat16),
        ],
        compiler_params=pltpu.CompilerParams(
            vmem_limit_bytes=110 * 1024 * 1024),
    )(off, xs, W1, b1r, W2, b2r)

    yg = gather(y, sp_rows)

    CT = 512
    out = pl.pallas_call(
        _combine_body,
        grid=(T // CT,),
        in_specs=[
            pl.BlockSpec((CT, D), lambda c: (c, 0)),
            pl.BlockSpec((CT, D), lambda c, nb=T // CT: (c + nb, 0)),
            pl.BlockSpec((CT, E), lambda c: (c, 0)),
        ],
        out_specs=pl.BlockSpec((CT, D), lambda c: (c, 0)),
        out_shape=jax.ShapeDtypeStruct((T, D), jnp.float32),
    )(yg, yg, wc)

    return out.reshape(b, s, d), aux.reshape(())
